# baseline (speedup 1.0000x reference)
import functools

import jax
import jax.numpy as jnp
from jax import lax
from jax.experimental import pallas as pl
from jax.experimental.pallas import tpu as pltpu
from jax.experimental.pallas import tpu_sc as plsc

N = 10000
E = 320000
D = 128
NS = 16
NPAD = 10240
EPT = E // NS
K = 128
NCHUNK = (EPT + K - 1) // K
EPT_P = NCHUNK * K
NACC = 10112
ACC_RPT = NACC // NS
RB = 3
ID = 4
SUP = 12
BM = 512

_mesh = plsc.VectorSubcoreMesh(core_axis_name="c", subcore_axis_name="s",
                               num_cores=2, num_subcores=NS)


def _zero_fill(buf):
    @pl.loop(0, K)
    def _f(i):
        for j in range(D // 16):
            buf[i, pl.ds(j * 16, 16)] = jnp.zeros((16,), jnp.float32)


def _zero_accum(accum, zb, tid):
    base = tid * ACC_RPT
    for i in range(ACC_RPT // K):
        pltpu.sync_copy(zb, accum.at[pl.ds(base + i * K, K), :])
    rem = ACC_RPT % K
    pltpu.sync_copy(zb.at[pl.ds(0, rem), :],
                    accum.at[pl.ds(base + ACC_RPT - rem, rem), :])


@functools.partial(
    pl.kernel,
    out_type=jax.ShapeDtypeStruct((2, NPAD, D), jnp.float32),
    mesh=_mesh,
    scratch_types=[
        pltpu.VMEM((K, D), jnp.float32),
        [pltpu.VMEM((2, K), jnp.int32)] * ID,
        [pltpu.SemaphoreType.DMA] * ID,
        [pltpu.SemaphoreType.DMA] * ID,
        pltpu.VMEM_SHARED((NACC, D), jnp.float32),
    ],
)
def _deg_kernel(e_all, deg_all, ones_v, didx, isem, ssem, accum):
    cid = lax.axis_index("c")
    tid = lax.axis_index("s")

    _zero_fill(ones_v)
    _zero_accum(accum, ones_v, tid)

    @pl.loop(0, K)
    def _fill_o(i):
        for j in range(D // 16):
            ones_v[i, pl.ds(j * 16, 16)] = jnp.ones((16,), jnp.float32)

    def stage_idx(c, j):
        pltpu.async_copy(e_all.at[cid, tid, c], didx[j], isem[j])

    def wait_idx(j):
        pltpu.make_async_copy(e_all.at[cid, tid, 0], didx[j], isem[j]).wait()

    def fire_scatter(j):
        pltpu.async_copy(ones_v, accum.at[didx[j].at[1]], ssem[j], add=True)

    def wait_scatter(j):
        pltpu.make_async_copy(ones_v, accum.at[didx[0].at[1]],
                              ssem[j]).wait()

    for j in range(2):
        stage_idx(j, j)

    plsc.subcore_barrier()

    def body(c, k):
        j = k % ID

        @pl.when(c >= 2)
        def _():
            wait_scatter((k + 2) % ID)

        @pl.when(c + 2 < NCHUNK)
        def _():
            stage_idx(c + 2, (k + 2) % ID)

        wait_idx(j)
        fire_scatter(j)

    G = NCHUNK // ID

    @pl.loop(0, G)
    def _steady(g):
        for k in range(ID):
            body(g * ID + k, k)

    for c in range(G * ID, NCHUNK):
        body(c, c % ID)
    for c in range(NCHUNK - 2, NCHUNK):
        wait_scatter(c % ID)

    plsc.subcore_barrier()
    sl = pl.ds(tid * ACC_RPT, ACC_RPT)
    pltpu.sync_copy(accum.at[sl, :], deg_all.at[cid, sl, :])


@functools.partial(
    pl.kernel,
    out_type=jax.ShapeDtypeStruct((2, NPAD, D), jnp.float32),
    mesh=_mesh,
    scratch_types=[
        [pltpu.VMEM((2, K), jnp.int32)] * ID,
        [pltpu.VMEM((K, D), jnp.float32)] * RB,
        [pltpu.SemaphoreType.DMA] * ID,
        [pltpu.SemaphoreType.DMA] * RB,
        [pltpu.SemaphoreType.DMA] * RB,
        pltpu.VMEM_SHARED((NACC, D), jnp.float32),
    ],
)
def _conv_kernel(h_all, e_all, conv_all, eidx, rows, isem, gsem, ssem, accum):
    cid = lax.axis_index("c")
    tid = lax.axis_index("s")
    h_hbm = h_all.at[cid]

    _zero_fill(rows[0])
    _zero_accum(accum, rows[0], tid)

    def stage_idx(c, j):
        pltpu.async_copy(e_all.at[cid, tid, c], eidx[j], isem[j])

    def wait_idx(j):
        pltpu.make_async_copy(e_all.at[cid, tid, 0], eidx[j], isem[j]).wait()

    def fire_gather(j, b):
        pltpu.async_copy(h_hbm.at[eidx[j].at[0]], rows[b], gsem[b])

    def wait_gather(b):
        pltpu.make_async_copy(h_hbm.at[eidx[0].at[0]], rows[b],
                              gsem[b]).wait()

    def fire_scatter(j, b):
        pltpu.async_copy(rows[b], accum.at[eidx[j].at[1]], ssem[b], add=True)

    def wait_scatter(b):
        pltpu.make_async_copy(rows[b], accum.at[eidx[0].at[1]],
                              ssem[b]).wait()

    for j in range(2):
        stage_idx(j, j)
    wait_idx(0)
    fire_gather(0, 0)

    plsc.subcore_barrier()

    def body(c, k):
        j = k % ID
        b = k % RB

        @pl.when(c + 1 < NCHUNK)
        def _():
            @pl.when(c >= RB - 1)
            def _():
                wait_scatter((k + 1) % RB)
            wait_idx((k + 1) % ID)
            fire_gather((k + 1) % ID, (k + 1) % RB)

        @pl.when(c + 2 < NCHUNK)
        def _():
            stage_idx(c + 2, (k + 2) % ID)

        wait_gather(b)
        fire_scatter(j, b)

    G = NCHUNK // SUP

    @pl.loop(0, G)
    def _steady(g):
        for k in range(SUP):
            body(g * SUP + k, k)

    for c in range(G * SUP, NCHUNK):
        body(c, c % SUP)
    for b in range(RB):
        wait_scatter(b)

    plsc.subcore_barrier()
    sl = pl.ds(tid * ACC_RPT, ACC_RPT)
    pltpu.sync_copy(accum.at[sl, :], conv_all.at[cid, sl, :])


def _dinv(deg_col):
    return jnp.where(deg_col > 0, lax.rsqrt(jnp.maximum(deg_col, 1e-12)), 0.0)


def _mm_body(x_ref, hw_ref, sw_ref, iw_ref, deg_ref, hh_ref, hs_ref, hi_ref):
    x = x_ref[...]
    du = _dinv(deg_ref[0, :, 0:1])
    dd = _dinv(deg_ref[1, :, 0:1])
    hh_ref[...] = lax.dot_general(x, hw_ref[...], (((1,), (1,)), ((), ())),
                                  preferred_element_type=jnp.float32)
    hs_ref[0] = du * lax.dot_general(x, sw_ref[...], (((1,), (0,)), ((), ())),
                                     preferred_element_type=jnp.float32)
    hi_ref[0] = dd * lax.dot_general(x, iw_ref[...], (((1,), (0,)), ((), ())),
                                     preferred_element_type=jnp.float32)


def _combine_mm_body(hh1_ref, conv_ref, deg_ref, hb_ref, sb_ref, ib_ref,
                     hw2_ref, sw2_ref, iw2_ref, hh_ref, hs_ref, hi_ref):
    du = _dinv(deg_ref[0, :, 0:1])
    dd = _dinv(deg_ref[1, :, 0:1])
    x2 = jnp.maximum(
        hh1_ref[...] + hb_ref[...] + du * conv_ref[0] + sb_ref[...]
        + dd * conv_ref[1] + ib_ref[...], 0.0)
    hh_ref[...] = lax.dot_general(x2, hw2_ref[...], (((1,), (1,)), ((), ())),
                                  preferred_element_type=jnp.float32)
    hs_ref[0] = du * lax.dot_general(x2, sw2_ref[...], (((1,), (0,)), ((), ())),
                                     preferred_element_type=jnp.float32)
    hi_ref[0] = dd * lax.dot_general(x2, iw2_ref[...], (((1,), (0,)), ((), ())),
                                     preferred_element_type=jnp.float32)


def _combine_body(hh_ref, conv_ref, deg_ref, hb_ref, sb_ref, ib_ref, out_ref):
    du = _dinv(deg_ref[0, :, 0:1])
    dd = _dinv(deg_ref[1, :, 0:1])
    out_ref[...] = jnp.maximum(
        hh_ref[...] + hb_ref[...] + du * conv_ref[0] + sb_ref[...]
        + dd * conv_ref[1] + ib_ref[...], 0.0)


_row_spec = pl.BlockSpec((BM, D), lambda i: (i, 0))
_pair_spec = pl.BlockSpec((2, BM, D), lambda i: (0, i, 0))
_half_spec = pl.BlockSpec((1, BM, D), lambda i: (0, i, 0))
_w_spec = pl.BlockSpec((D, D), lambda i: (0, 0))
_b_spec = pl.BlockSpec((1, D), lambda i: (0, 0))
_GRID = (NPAD // BM,)
_sds = jax.ShapeDtypeStruct((NPAD, D), jnp.float32)
_hsds = jax.ShapeDtypeStruct((1, NPAD, D), jnp.float32)

_mm_call = pl.pallas_call(
    _mm_body,
    grid=_GRID,
    in_specs=[_row_spec, _w_spec, _w_spec, _w_spec, _pair_spec],
    out_specs=[_row_spec, _half_spec, _half_spec],
    out_shape=[_sds, _hsds, _hsds],
)

_combine_mm_call = pl.pallas_call(
    _combine_mm_body,
    grid=_GRID,
    in_specs=[_row_spec, _pair_spec, _pair_spec, _b_spec, _b_spec, _b_spec,
              _w_spec, _w_spec, _w_spec],
    out_specs=[_row_spec, _half_spec, _half_spec],
    out_shape=[_sds, _hsds, _hsds],
)

_combine_call = pl.pallas_call(
    _combine_body,
    grid=_GRID,
    in_specs=[_row_spec, _pair_spec, _pair_spec, _b_spec, _b_spec, _b_spec],
    out_specs=_row_spec,
    out_shape=_sds,
)


def _pad_edges(row, fill):
    r = row.reshape(NS, EPT)
    r = jnp.pad(r, ((0, 0), (0, EPT_P - EPT)), constant_values=fill)
    return r.reshape(NS, NCHUNK, K)


def kernel(x, Lu, Ld, har_w1, har_b1, sol_w1, sol_b1, irr_w1, irr_b1,
           har_w2, har_b2, sol_w2, sol_b2, irr_w2, irr_b2):
    xp = jnp.pad(x, ((0, NPAD - N), (0, 0)))
    e_all = jnp.stack([
        jnp.stack([_pad_edges(Lu[0], 0), _pad_edges(Lu[1], N)], axis=2),
        jnp.stack([_pad_edges(Ld[0], 0), _pad_edges(Ld[1], N)], axis=2),
    ])
    hb1, sb1, ib1 = har_b1[None, :], sol_b1[None, :], irr_b1[None, :]
    hb2, sb2, ib2 = har_b2[None, :], sol_b2[None, :], irr_b2[None, :]

    deg_all = _deg_kernel(e_all)
    hh1, hs1, hi1 = _mm_call(xp, har_w1, sol_w1, irr_w1, deg_all)
    h_all1 = jnp.concatenate([hs1, hi1], axis=0)
    conv1 = _conv_kernel(h_all1, e_all)
    hh2, hs2, hi2 = _combine_mm_call(hh1, conv1, deg_all, hb1, sb1, ib1,
                                     har_w2, sol_w2, irr_w2)
    h_all2 = jnp.concatenate([hs2, hi2], axis=0)
    conv2 = _conv_kernel(h_all2, e_all)
    out = _combine_call(hh2, conv2, deg_all, hb2, sb2, ib2)
    return out[:N]


# speedup vs baseline: 1.0210x; 1.0210x over previous
import functools

import jax
import jax.numpy as jnp
from jax import lax
from jax.experimental import pallas as pl
from jax.experimental.pallas import tpu as pltpu
from jax.experimental.pallas import tpu_sc as plsc

N = 10000
E = 320000
D = 128
NS = 16
NPAD = 10240
EPT = E // NS
K = 128
NCHUNK = (EPT + K - 1) // K
EPT_P = NCHUNK * K
NACC = 10112
ACC_RPT = NACC // NS
RB = 3
ID = 4
SUP = 12
BM = 512

_mesh = plsc.VectorSubcoreMesh(core_axis_name="c", subcore_axis_name="s",
                               num_cores=2, num_subcores=NS)


def _zero_fill(buf):
    @pl.loop(0, K)
    def _f(i):
        for j in range(D // 16):
            buf[i, pl.ds(j * 16, 16)] = jnp.zeros((16,), jnp.float32)


def _zero_accum(accum, zb, tid):
    base = tid * ACC_RPT
    for i in range(ACC_RPT // K):
        pltpu.sync_copy(zb, accum.at[pl.ds(base + i * K, K), :])
    rem = ACC_RPT % K
    pltpu.sync_copy(zb.at[pl.ds(0, rem), :],
                    accum.at[pl.ds(base + ACC_RPT - rem, rem), :])


@functools.partial(
    pl.kernel,
    out_type=jax.ShapeDtypeStruct((2, NPAD, D), jnp.float32),
    mesh=_mesh,
    scratch_types=[
        pltpu.VMEM((K, D), jnp.float32),
        [pltpu.VMEM((K,), jnp.int32)] * ID,
        [pltpu.SemaphoreType.DMA] * ID,
        [pltpu.SemaphoreType.DMA] * ID,
        pltpu.VMEM_SHARED((NACC, D), jnp.float32),
    ],
)
def _deg_kernel(dst_all, deg_all, ones_v, didx, isem, ssem, accum):
    cid = lax.axis_index("c")
    tid = lax.axis_index("s")

    _zero_fill(ones_v)
    _zero_accum(accum, ones_v, tid)

    @pl.loop(0, K)
    def _fill_o(i):
        for j in range(D // 16):
            ones_v[i, pl.ds(j * 16, 16)] = jnp.ones((16,), jnp.float32)

    def stage_idx(c, j):
        pltpu.async_copy(dst_all.at[cid, tid, c], didx[j], isem[j])

    def wait_idx(j):
        pltpu.make_async_copy(dst_all.at[cid, tid, 0], didx[j],
                              isem[j]).wait()

    def fire_scatter(j):
        pltpu.async_copy(ones_v, accum.at[didx[j]], ssem[j], add=True)

    def wait_scatter(j):
        pltpu.make_async_copy(ones_v, accum.at[didx[0]], ssem[j]).wait()

    for j in range(2):
        stage_idx(j, j)

    plsc.subcore_barrier()

    def body(c, k):
        j = k % ID

        @pl.when(c >= 2)
        def _():
            wait_scatter((k + 2) % ID)

        @pl.when(c + 2 < NCHUNK)
        def _():
            stage_idx(c + 2, (k + 2) % ID)

        wait_idx(j)
        fire_scatter(j)

    G = NCHUNK // ID

    @pl.loop(0, G)
    def _steady(g):
        for k in range(ID):
            body(g * ID + k, k)

    for c in range(G * ID, NCHUNK):
        body(c, c % ID)
    for c in range(NCHUNK - 2, NCHUNK):
        wait_scatter(c % ID)

    plsc.subcore_barrier()
    sl = pl.ds(tid * ACC_RPT, ACC_RPT)
    pltpu.sync_copy(accum.at[sl, :], deg_all.at[cid, sl, :])


@functools.partial(
    pl.kernel,
    out_type=jax.ShapeDtypeStruct((2, NPAD, D), jnp.float32),
    mesh=_mesh,
    scratch_types=[
        [pltpu.VMEM((2, K), jnp.int32)] * ID,
        [pltpu.VMEM((K, D), jnp.float32)] * RB,
        [pltpu.SemaphoreType.DMA] * ID,
        [pltpu.SemaphoreType.DMA] * RB,
        [pltpu.SemaphoreType.DMA] * RB,
        pltpu.VMEM_SHARED((NACC, D), jnp.float32),
    ],
)
def _conv_kernel(h_all, e_all, conv_all, eidx, rows, isem, gsem, ssem, accum):
    cid = lax.axis_index("c")
    tid = lax.axis_index("s")
    h_hbm = h_all.at[cid]

    _zero_fill(rows[0])
    _zero_accum(accum, rows[0], tid)

    def stage_idx(c, j):
        pltpu.async_copy(e_all.at[cid, tid, c], eidx[j], isem[j])

    def wait_idx(j):
        pltpu.make_async_copy(e_all.at[cid, tid, 0], eidx[j], isem[j]).wait()

    def fire_gather(j, b):
        pltpu.async_copy(h_hbm.at[eidx[j].at[0]], rows[b], gsem[b])

    def wait_gather(b):
        pltpu.make_async_copy(h_hbm.at[eidx[0].at[0]], rows[b],
                              gsem[b]).wait()

    def fire_scatter(j, b):
        pltpu.async_copy(rows[b], accum.at[eidx[j].at[1]], ssem[b], add=True)

    def wait_scatter(b):
        pltpu.make_async_copy(rows[b], accum.at[eidx[0].at[1]],
                              ssem[b]).wait()

    for j in range(2):
        stage_idx(j, j)
    wait_idx(0)
    fire_gather(0, 0)

    plsc.subcore_barrier()

    def body(c, k):
        j = k % ID
        b = k % RB

        @pl.when(c + 1 < NCHUNK)
        def _():
            @pl.when(c >= RB - 1)
            def _():
                wait_scatter((k + 1) % RB)
            wait_idx((k + 1) % ID)
            fire_gather((k + 1) % ID, (k + 1) % RB)

        @pl.when(c + 2 < NCHUNK)
        def _():
            stage_idx(c + 2, (k + 2) % ID)

        wait_gather(b)
        fire_scatter(j, b)

    G = NCHUNK // SUP

    @pl.loop(0, G)
    def _steady(g):
        for k in range(SUP):
            body(g * SUP + k, k)

    for c in range(G * SUP, NCHUNK):
        body(c, c % SUP)
    for b in range(RB):
        wait_scatter(b)

    plsc.subcore_barrier()
    sl = pl.ds(tid * ACC_RPT, ACC_RPT)
    pltpu.sync_copy(accum.at[sl, :], conv_all.at[cid, sl, :])


def _dinv(deg_col):
    return jnp.where(deg_col > 0, lax.rsqrt(jnp.maximum(deg_col, 1e-12)), 0.0)


def _mm_body(x_ref, hw_ref, sw_ref, iw_ref, deg_ref, hh_ref, hs_ref, hi_ref):
    x = x_ref[...]
    du = _dinv(deg_ref[0, :, 0:1])
    dd = _dinv(deg_ref[1, :, 0:1])
    hh_ref[...] = lax.dot_general(x, hw_ref[...], (((1,), (1,)), ((), ())),
                                  preferred_element_type=jnp.float32)
    hs_ref[0] = du * lax.dot_general(x, sw_ref[...], (((1,), (0,)), ((), ())),
                                     preferred_element_type=jnp.float32)
    hi_ref[0] = dd * lax.dot_general(x, iw_ref[...], (((1,), (0,)), ((), ())),
                                     preferred_element_type=jnp.float32)


def _combine_mm_body(hh1_ref, conv_ref, deg_ref, hb_ref, sb_ref, ib_ref,
                     hw2_ref, sw2_ref, iw2_ref, hh_ref, hs_ref, hi_ref):
    du = _dinv(deg_ref[0, :, 0:1])
    dd = _dinv(deg_ref[1, :, 0:1])
    x2 = jnp.maximum(
        hh1_ref[...] + hb_ref[...] + du * conv_ref[0] + sb_ref[...]
        + dd * conv_ref[1] + ib_ref[...], 0.0)
    hh_ref[...] = lax.dot_general(x2, hw2_ref[...], (((1,), (1,)), ((), ())),
                                  preferred_element_type=jnp.float32)
    hs_ref[0] = du * lax.dot_general(x2, sw2_ref[...], (((1,), (0,)), ((), ())),
                                     preferred_element_type=jnp.float32)
    hi_ref[0] = dd * lax.dot_general(x2, iw2_ref[...], (((1,), (0,)), ((), ())),
                                     preferred_element_type=jnp.float32)


def _combine_body(hh_ref, conv_ref, deg_ref, hb_ref, sb_ref, ib_ref, out_ref):
    du = _dinv(deg_ref[0, :, 0:1])
    dd = _dinv(deg_ref[1, :, 0:1])
    out_ref[...] = jnp.maximum(
        hh_ref[...] + hb_ref[...] + du * conv_ref[0] + sb_ref[...]
        + dd * conv_ref[1] + ib_ref[...], 0.0)


_row_spec = pl.BlockSpec((BM, D), lambda i: (i, 0))
_pair_spec = pl.BlockSpec((2, BM, D), lambda i: (0, i, 0))
_half_spec = pl.BlockSpec((1, BM, D), lambda i: (0, i, 0))
_w_spec = pl.BlockSpec((D, D), lambda i: (0, 0))
_b_spec = pl.BlockSpec((1, D), lambda i: (0, 0))
_GRID = (NPAD // BM,)
_sds = jax.ShapeDtypeStruct((NPAD, D), jnp.float32)
_hsds = jax.ShapeDtypeStruct((1, NPAD, D), jnp.float32)

_mm_call = pl.pallas_call(
    _mm_body,
    grid=_GRID,
    in_specs=[_row_spec, _w_spec, _w_spec, _w_spec, _pair_spec],
    out_specs=[_row_spec, _half_spec, _half_spec],
    out_shape=[_sds, _hsds, _hsds],
)

_combine_mm_call = pl.pallas_call(
    _combine_mm_body,
    grid=_GRID,
    in_specs=[_row_spec, _pair_spec, _pair_spec, _b_spec, _b_spec, _b_spec,
              _w_spec, _w_spec, _w_spec],
    out_specs=[_row_spec, _half_spec, _half_spec],
    out_shape=[_sds, _hsds, _hsds],
)

_combine_call = pl.pallas_call(
    _combine_body,
    grid=_GRID,
    in_specs=[_row_spec, _pair_spec, _pair_spec, _b_spec, _b_spec, _b_spec],
    out_specs=_row_spec,
    out_shape=_sds,
)


def _pad_edges(row, fill):
    r = row.reshape(NS, EPT)
    r = jnp.pad(r, ((0, 0), (0, EPT_P - EPT)), constant_values=fill)
    return r.reshape(NS, NCHUNK, K)


def kernel(x, Lu, Ld, har_w1, har_b1, sol_w1, sol_b1, irr_w1, irr_b1,
           har_w2, har_b2, sol_w2, sol_b2, irr_w2, irr_b2):
    xp = jnp.pad(x, ((0, NPAD - N), (0, 0)))
    dst_all = jnp.stack([_pad_edges(Lu[1], N), _pad_edges(Ld[1], N)])
    e_all = jnp.stack([
        jnp.stack([_pad_edges(Lu[0], 0), _pad_edges(Lu[1], N)], axis=2),
        jnp.stack([_pad_edges(Ld[0], 0), _pad_edges(Ld[1], N)], axis=2),
    ])
    hb1, sb1, ib1 = har_b1[None, :], sol_b1[None, :], irr_b1[None, :]
    hb2, sb2, ib2 = har_b2[None, :], sol_b2[None, :], irr_b2[None, :]

    deg_all = _deg_kernel(dst_all)
    hh1, hs1, hi1 = _mm_call(xp, har_w1, sol_w1, irr_w1, deg_all)
    h_all1 = jnp.concatenate([hs1, hi1], axis=0)
    conv1 = _conv_kernel(h_all1, e_all)
    hh2, hs2, hi2 = _combine_mm_call(hh1, conv1, deg_all, hb1, sb1, ib1,
                                     har_w2, sol_w2, irr_w2)
    h_all2 = jnp.concatenate([hs2, hi2], axis=0)
    conv2 = _conv_kernel(h_all2, e_all)
    out = _combine_call(hh2, conv2, deg_all, hb2, sb2, ib2)
    return out[:N]
